# Initial kernel scaffold; baseline (speedup 1.0000x reference)
import jax
import jax.numpy as jnp
from jax.experimental import pallas as pl


def kernel(data, edge_index, batch, W0, b0, Wl, bl, fc_w, fc_b, Wa, va, Wp, conv1_w, conv1_b, conv2_w, conv2_b, fcc_w, fcc_b):
    raise NotImplementedError("write your pallas kernel here")


# speedup vs baseline: 1.6414x; 1.6414x over previous
import jax
import jax.numpy as jnp
from jax import lax
from jax.experimental import pallas as pl

_F32 = jnp.float32
_NPAD = 10240
_BLK = 2048


def _l1_body(x, w, b, dn, g):
    acc = jnp.dot(x[...], w[...], preferred_element_type=_F32) + b[...]
    g[...] = acc * dn[...]


def _layer1(x, w, b2d, dn2d):
    n, k = x.shape
    h = w.shape[1]
    return pl.pallas_call(
        _l1_body,
        grid=(n // _BLK,),
        in_specs=[
            pl.BlockSpec((_BLK, k), lambda i: (i, 0)),
            pl.BlockSpec((k, h), lambda i: (0, 0)),
            pl.BlockSpec((1, h), lambda i: (0, 0)),
            pl.BlockSpec((_BLK, 1), lambda i: (i, 0)),
        ],
        out_specs=pl.BlockSpec((_BLK, h), lambda i: (i, 0)),
        out_shape=jax.ShapeDtypeStruct((n, h), _F32),
    )(x, w, b2d, dn2d)


def _l2_body(s, g, dn, w, b, o):
    d = dn[...]
    h = jnp.maximum(d * (s[...] + g[...]), 0.0)
    hh = jnp.dot(h, w[...], preferred_element_type=_F32) + b[...]
    o[...] = hh * d


def _layer2(s, g, dn2d, w, b2d):
    n, h = g.shape
    return pl.pallas_call(
        _l2_body,
        grid=(n // _BLK,),
        in_specs=[
            pl.BlockSpec((_BLK, h), lambda i: (i, 0)),
            pl.BlockSpec((_BLK, h), lambda i: (i, 0)),
            pl.BlockSpec((_BLK, 1), lambda i: (i, 0)),
            pl.BlockSpec((h, h), lambda i: (0, 0)),
            pl.BlockSpec((1, h), lambda i: (0, 0)),
        ],
        out_specs=pl.BlockSpec((_BLK, h), lambda i: (i, 0)),
        out_shape=jax.ShapeDtypeStruct((n, h), _F32),
    )(s, g, dn2d, w, b2d)


def _zc_body(s, g, dn, wa, va, z, sc):
    d = dn[...]
    h = jnp.maximum(d * (s[...] + g[...]), 0.0)
    z[...] = h
    t = jnp.tanh(jnp.dot(h, wa[...], preferred_element_type=_F32))
    sc[...] = jnp.sum(t * va[...], axis=1, keepdims=True)


def _final_z(s, g, dn2d, wa, va_row):
    n, h = g.shape
    dkv = wa.shape[1]
    return pl.pallas_call(
        _zc_body,
        grid=(n // _BLK,),
        in_specs=[
            pl.BlockSpec((_BLK, h), lambda i: (i, 0)),
            pl.BlockSpec((_BLK, h), lambda i: (i, 0)),
            pl.BlockSpec((_BLK, 1), lambda i: (i, 0)),
            pl.BlockSpec((h, dkv), lambda i: (0, 0)),
            pl.BlockSpec((1, dkv), lambda i: (0, 0)),
        ],
        out_specs=[pl.BlockSpec((_BLK, h), lambda i: (i, 0)),
                   pl.BlockSpec((_BLK, 1), lambda i: (i, 0))],
        out_shape=[jax.ShapeDtypeStruct((n, h), _F32),
                   jax.ShapeDtypeStruct((n, 1), _F32)],
    )(s, g, dn2d, wa, va_row)


def _pool_body(oh, sc0, sc1, sc2, z0, z1, z2,
               n0, n1, n2, d0, d1, d2):
    i = pl.program_id(0)
    outs = (n0, n1, n2, d0, d1, d2)

    @pl.when(i == 0)
    def _():
        for o in outs:
            o[...] = jnp.zeros_like(o[...])

    ohv = oh[...]
    dims = (((0,), (0,)), ((), ()))

    def accum(nref, dref, scref, zref):
        ex = jnp.exp(scref[...])
        nref[...] += lax.dot_general(ohv, zref[...] * ex, dims,
                                     preferred_element_type=_F32)
        exb = jnp.broadcast_to(ex, (ex.shape[0], 128))
        dref[...] += lax.dot_general(ohv, exb, dims,
                                     preferred_element_type=_F32)

    accum(n0, d0, sc0, z0)
    accum(n1, d1, sc1, z1)
    accum(n2, d2, sc2, z2)


def _attn_pool(oh, scs, zs, nb):
    n, h = zs[0].shape
    spec_z = pl.BlockSpec((_BLK, h), lambda i: (i, 0))
    spec_sc = pl.BlockSpec((_BLK, 1), lambda i: (i, 0))
    return pl.pallas_call(
        _pool_body,
        grid=(n // _BLK,),
        in_specs=[pl.BlockSpec((_BLK, nb), lambda i: (i, 0))]
        + [spec_sc] * 3 + [spec_z] * 3,
        out_specs=[pl.BlockSpec((nb, h), lambda i: (0, 0))] * 3
        + [pl.BlockSpec((nb, 128), lambda i: (0, 0))] * 3,
        out_shape=[jax.ShapeDtypeStruct((nb, h), _F32)] * 3
        + [jax.ShapeDtypeStruct((nb, 128), _F32)] * 3,
    )(oh, *scs, *zs)


def kernel(data, edge_index, batch, W0, b0, Wl, bl, fc_w, fc_b, Wa, va, Wp,
           conv1_w, conv1_b, conv2_w, conv2_b, fcc_w, fcc_b):
    n_real, in_dim = data.shape
    e = edge_index.shape[1]
    sz_c, _, h = W0.shape
    dkv = va.shape[0]
    nb = 16

    n = _NPAD
    data = jnp.concatenate(
        [data, jnp.zeros((n - n_real, in_dim), _F32)], axis=0)
    batch = jnp.concatenate(
        [batch, jnp.full((n - n_real,), nb, batch.dtype)])

    src = edge_index[0]
    dst = edge_index[1]

    deg = jax.ops.segment_sum(jnp.ones((e,), _F32), dst, num_segments=n)
    dn = lax.rsqrt(deg + 1.0)
    dn2d = dn[:, None]

    va_row = va[None, :]

    def seg(g):
        return jax.ops.segment_sum(g[src], dst, num_segments=n)

    zs = []
    scs = []
    for c in range(sz_c):
        g = _layer1(data, W0[c], b0[c][None, :], dn2d)
        g = _layer2(seg(g), g, dn2d, Wl[c, 0], bl[c, 0][None, :])
        z, sc = _final_z(seg(g), g, dn2d, Wa, va_row)
        zs.append(z)
        scs.append(sc)

    oh = (batch[:, None] == jnp.arange(nb, dtype=batch.dtype)[None, :])
    oh = oh.astype(_F32)
    n0, n1, n2, d0, d1, d2 = _attn_pool(oh, scs, zs, nb)
    pooled = jnp.stack([n0 / d0[:, :1], n1 / d1[:, :1], n2 / d2[:, :1]],
                       axis=0)

    emb = jnp.transpose(pooled, (1, 0, 2)) @ Wp
    img = emb.reshape(nb, sz_c, dkv, dkv)

    def conv(x, w, b):
        y = lax.conv_general_dilated(x, w, window_strides=(1, 1),
                                     padding='VALID',
                                     dimension_numbers=('NCHW', 'OIHW', 'NCHW'))
        return y + b[None, :, None, None]

    def pool2(x):
        return lax.reduce_window(x, -jnp.inf, lax.max, (1, 1, 2, 2),
                                 (1, 1, 2, 2), 'VALID')

    c1_ = pool2(jax.nn.relu(conv(img, conv1_w, conv1_b)))
    c2_ = pool2(jax.nn.relu(conv(c1_, conv2_w, conv2_b)))
    flat = c2_.reshape(nb, -1)
    logits = flat @ fcc_w + fcc_b
    return jax.nn.log_softmax(logits, axis=-1)
